# Initial kernel scaffold; baseline (speedup 1.0000x reference)
import jax
import jax.numpy as jnp
from jax.experimental import pallas as pl


def kernel(feat, proposals, image_shape, W6, b6, W7, b7, Wc, bc, Wb, bb):
    raise NotImplementedError("write your pallas kernel here")


# speedup vs baseline: 5.6028x; 5.6028x over previous
import math

import jax
import jax.numpy as jnp
from jax.experimental import pallas as pl
from jax.experimental.pallas import tpu as pltpu

_NEG = -1e30
_H = 25
_W = 25
_C = 512
_P = 7
_K = 6
_RT = 8
_LOG_MAX = math.log(1000.0 / 16)


def _pool_kernel(idx_ref, f_ref, o_ref, rm_ref):
    def per_roi(r, carry):
        for ph in range(_P):
            s = idx_ref[r, ph]
            e = idx_ref[r, _P + ph]
            acc = jnp.full((_W, _C), _NEG, dtype=jnp.float32)
            for k in range(_K):
                h = s + k
                hc = jnp.clip(h, 0, _H - 1)
                row = f_ref[hc]
                acc = jnp.where(h < e, jnp.maximum(acc, row), acc)
            rm_ref[:, ph, :] = acc
        for pw in range(_P):
            s = idx_ref[r, 2 * _P + pw]
            e = idx_ref[r, 3 * _P + pw]
            acc = jnp.full((_P, _C), _NEG, dtype=jnp.float32)
            for k in range(_K):
                w = s + k
                wc = jnp.clip(w, 0, _W - 1)
                col = rm_ref[wc]
                acc = jnp.where(w < e, jnp.maximum(acc, col), acc)
            acc = jnp.where(acc <= _NEG * 0.5, 0.0, acc)
            o_ref[r, pl.ds(pw * _P, _P), :] = acc
        return carry

    jax.lax.fori_loop(0, _RT, per_roi, 0)


def _fc1_kernel(x_ref, w_ref, b_ref, o_ref):
    k = pl.program_id(0)

    @pl.when(k == 0)
    def _():
        o_ref[...] = jnp.zeros_like(o_ref)

    o_ref[...] += jnp.dot(x_ref[...], w_ref[...],
                          preferred_element_type=jnp.float32)

    @pl.when(k == pl.num_programs(0) - 1)
    def _():
        o_ref[...] = jnp.maximum(o_ref[...] + b_ref[...], 0.0)


def _head_kernel(h1_ref, w7_ref, b7_ref, wh_ref, bh_ref, pr_ref, img_ref,
                 o_ref):
    h2 = jnp.maximum(
        jnp.dot(h1_ref[...], w7_ref[...], preferred_element_type=jnp.float32)
        + b7_ref[...], 0.0)
    y = jnp.dot(h2, wh_ref[...], preferred_element_type=jnp.float32) \
        + bh_ref[...]
    scores = y[:, 0:21]
    bp = y[:, 21:105]

    x1 = pr_ref[:, 0:1]
    y1 = pr_ref[:, 1:2]
    x2 = pr_ref[:, 2:3]
    y2 = pr_ref[:, 3:4]
    w = x2 - x1
    h = y2 - y1
    cx = x1 + 0.5 * w
    cy = y1 + 0.5 * h

    comp = jax.lax.broadcasted_iota(jnp.int32, bp.shape, 1) % 4
    even = (comp % 2) == 0
    s = jnp.where(even, w, h)
    t = jnp.where(even, cx, cy)
    bsh = jnp.concatenate([bp[:, 2:], bp[:, :2]], axis=1)
    csh = jnp.concatenate([bp[:, 82:], bp[:, :82]], axis=1)
    sz1 = jnp.exp(jnp.minimum(bsh, _LOG_MAX)) * s * 0.5
    sz2 = jnp.exp(jnp.minimum(bp, _LOG_MAX)) * s * 0.5
    xy1 = bp * s + t - sz1
    xy2 = csh * s + t + sz2
    res = jnp.where(comp < 2, xy1, xy2)
    wi = img_ref[1].astype(jnp.float32)
    hi = img_ref[0].astype(jnp.float32)
    bound = jnp.where(even, wi, hi)
    res = jnp.clip(res, 0.0, bound)
    o_ref[...] = jnp.concatenate([scores, res], axis=1)


def kernel(feat, proposals, image_shape, W6, b6, W7, b7, Wc, bc, Wb, bb):
    N = proposals.shape[0]
    f = jnp.transpose(feat[0], (1, 2, 0))

    scale = 1.0 / 16
    x1 = jnp.round(proposals[:, 0] * scale)
    y1 = jnp.round(proposals[:, 1] * scale)
    x2 = jnp.round(proposals[:, 2] * scale)
    y2 = jnp.round(proposals[:, 3] * scale)
    roi_w = jnp.maximum(x2 - x1 + 1.0, 1.0)
    roi_h = jnp.maximum(y2 - y1 + 1.0, 1.0)
    bh = roi_h / _P
    bw = roi_w / _P
    pidx = jnp.arange(_P, dtype=jnp.float32)
    hs = jnp.clip(jnp.floor(pidx[None, :] * bh[:, None]) + y1[:, None], 0, _H)
    he = jnp.clip(jnp.ceil((pidx[None, :] + 1.0) * bh[:, None]) + y1[:, None],
                  0, _H)
    ws = jnp.clip(jnp.floor(pidx[None, :] * bw[:, None]) + x1[:, None], 0, _W)
    we = jnp.clip(jnp.ceil((pidx[None, :] + 1.0) * bw[:, None]) + x1[:, None],
                  0, _W)
    idx = jnp.concatenate([hs, he, ws, we], axis=1).astype(jnp.int32)

    X = pl.pallas_call(
        _pool_kernel,
        grid=(N // _RT,),
        in_specs=[
            pl.BlockSpec((_RT, 4 * _P), lambda i: (i, 0),
                         memory_space=pltpu.SMEM),
            pl.BlockSpec((_H, _W, _C), lambda i: (0, 0, 0)),
        ],
        out_specs=pl.BlockSpec((_RT, _P * _P, _C), lambda i: (i, 0, 0)),
        out_shape=jax.ShapeDtypeStruct((N, _P * _P, _C), jnp.float32),
        scratch_shapes=[pltpu.VMEM((_W, _P, _C), jnp.float32)],
    )(idx, f)
    X2 = X.reshape(N, _P * _P * _C)

    d_in = _P * _P * _C
    W6p = (W6.reshape(1024, _C, _P, _P).transpose(3, 2, 1, 0)
           .reshape(d_in, 1024))
    NP = 1024
    KT = 512
    h1 = pl.pallas_call(
        _fc1_kernel,
        grid=(d_in // KT,),
        in_specs=[
            pl.BlockSpec((NP, KT), lambda k: (0, k)),
            pl.BlockSpec((KT, 1024), lambda k: (k, 0)),
            pl.BlockSpec((1, 1024), lambda k: (0, 0)),
        ],
        out_specs=pl.BlockSpec((NP, 1024), lambda k: (0, 0)),
        out_shape=jax.ShapeDtypeStruct((NP, 1024), jnp.float32),
    )(X2, W6p, b6.reshape(1, 1024))

    Wh = jnp.concatenate([Wc, Wb], axis=0)
    bh2 = jnp.concatenate([bc, bb]).reshape(1, 105)
    RT2 = 128
    out = pl.pallas_call(
        _head_kernel,
        grid=(NP // RT2,),
        in_specs=[
            pl.BlockSpec((RT2, 1024), lambda i: (i, 0)),
            pl.BlockSpec((1024, 1024), lambda i: (0, 0)),
            pl.BlockSpec((1, 1024), lambda i: (0, 0)),
            pl.BlockSpec((1024, 105), lambda i: (0, 0)),
            pl.BlockSpec((1, 105), lambda i: (0, 0)),
            pl.BlockSpec((RT2, 4), lambda i: (i, 0)),
            pl.BlockSpec(memory_space=pltpu.SMEM),
        ],
        out_specs=pl.BlockSpec((RT2, 105), lambda i: (i, 0)),
        out_shape=jax.ShapeDtypeStruct((N, 105), jnp.float32),
    )(h1, W7.T, b7.reshape(1, 1024), Wh.T, bh2, proposals, image_shape)
    return out
